# baseline (speedup 1.0000x reference)
import functools

import jax
import jax.numpy as jnp
from jax import lax
from jax.experimental import pallas as pl
from jax.experimental.pallas import tpu as pltpu
from jax.experimental.pallas import tpu_sc as plsc


def _tc_sims_kernel(cue_ref, patches_ref, sims_ref, inv_ref):
    cue = cue_ref[0]
    p = patches_ref[0]

    cue_n = cue / jnp.maximum(
        jnp.sqrt(jnp.sum(cue * cue, axis=1, keepdims=True)), 1e-12)
    inv = 1.0 / jnp.maximum(jnp.sqrt(jnp.sum(p * p, axis=1)), 1e-12)
    pn = p * inv[:, None]

    sims_ref[0] = jax.lax.dot_general(
        cue_n, pn, (((1,), (1,)), ((), ())),
        preferred_element_type=jnp.float32,
    )
    inv_ref[0, 0] = inv


def _dyn_gather(v, idx):
    return lax.gather(
        v, idx[:, None],
        dimension_numbers=lax.GatherDimensionNumbers(
            offset_dims=(), collapsed_slice_dims=(0,), start_index_map=(0,)),
        slice_sizes=(1,),
        mode=lax.GatherScatterMode.PROMISE_IN_BOUNDS)

def _sc_body(sims_hbm, inv_hbm, patches_hbm, out_hbm,
             sims_v, inv_v, idx_v, rows_v, acc_v, sem):
    nc = 2
    wid = lax.axis_index("s") * nc + lax.axis_index("c")

    def one_row(rep, carry):
        pair = wid + 32 * rep
        b = lax.div(pair, 5)
        k = lax.rem(pair, 5)
        r = b * 8 + k

        @pl.when(pair < 80)
        def _():
            pltpu.sync_copy(sims_hbm.at[r], sims_v)
            pltpu.sync_copy(inv_hbm.at[b], inv_v)

            lane = lax.iota(jnp.int32, 16)

            def chunk_step(j, c):
                cur_v, cur_i, cur_min = c
                v = sims_v[pl.ds(j * 16, 16)]

                def do_merge(_):
                    idx = lane + j * 16
                    dv, di = plsc.sort_key_val(v, idx, descending=True)
                    take = (dv > cur_v) | ((dv == cur_v) & (di < cur_i))
                    nv, ni = plsc.sort_key_val(jnp.where(take, dv, cur_v),
                                               jnp.where(take, di, cur_i))
                    return (nv, ni, jnp.min(nv))

                return lax.cond(jnp.max(v) > cur_min, do_merge,
                                lambda _: (cur_v, cur_i, cur_min), 0)

            cur_v, cur_i, _ = lax.fori_loop(
                0, 256, chunk_step,
                (jnp.full((16,), -2.0, jnp.float32),
                 jnp.zeros((16,), jnp.int32),
                 jnp.float32(-2.0)))

            cnt = jnp.zeros((16,), jnp.int32)
            for j in range(16):
                sel = jnp.full((16,), j, jnp.int32)
                vj = _dyn_gather(cur_v, sel)
                ij = _dyn_gather(cur_i, sel)
                beats = (vj > cur_v) | ((vj == cur_v) & (ij < cur_i))
                cnt = cnt + beats.astype(jnp.int32)
            _, top_i = plsc.sort_key_val(cnt, cur_i)
            inv9 = plsc.load_gather(inv_v, [top_i])
            wvec = inv9 * jnp.float32(1.0 / 9.0)

            idx_v[...] = top_i + b * 4096
            pltpu.async_copy(patches_hbm.at[idx_v], rows_v, sem).wait()

            def acc_chunk(c, carry):
                s = jnp.zeros((16,), jnp.float32)
                for j in range(9):
                    wj = _dyn_gather(wvec, jnp.full((16,), j, jnp.int32))
                    s = s + rows_v[j, pl.ds(c * 16, 16)] * wj
                acc_v[pl.ds(c * 16, 16)] = s
                return carry

            lax.fori_loop(0, 32, acc_chunk, 0)
            pltpu.sync_copy(acc_v, out_hbm.at[r])

        return carry

    lax.fori_loop(0, 3, one_row, 0)


def kernel(cue, patches):
    b, k, d = cue.shape
    n = patches.shape[1]
    cue_p = jnp.pad(cue, ((0, 0), (0, 8 - k), (0, 0)))

    sims, inv = pl.pallas_call(
        _tc_sims_kernel,
        grid=(b,),
        in_specs=[
            pl.BlockSpec((1, 8, d), lambda i: (i, 0, 0)),
            pl.BlockSpec((1, n, d), lambda i: (i, 0, 0)),
        ],
        out_specs=[
            pl.BlockSpec((1, 8, n), lambda i: (i, 0, 0)),
            pl.BlockSpec((1, 1, n), lambda i: (i, 0, 0)),
        ],
        out_shape=[
            jax.ShapeDtypeStruct((b, 8, n), jnp.float32),
            jax.ShapeDtypeStruct((b, 1, n), jnp.float32),
        ],
    )(cue_p, patches)

    sims_flat = sims.reshape(b * 8, n)
    inv_flat = inv.reshape(b, n)
    patches_flat = patches.reshape(b * n, d)

    mesh = plsc.VectorSubcoreMesh(core_axis_name="c", subcore_axis_name="s")
    sc = functools.partial(
        pl.kernel,
        mesh=mesh,
        out_type=jax.ShapeDtypeStruct((b * 8, d), jnp.float32),
        scratch_types=[
            pltpu.VMEM((n,), jnp.float32),
            pltpu.VMEM((n,), jnp.float32),
            pltpu.VMEM((16,), jnp.int32),
            pltpu.VMEM((16, d), jnp.float32),
            pltpu.VMEM((d,), jnp.float32),
            pltpu.SemaphoreType.DMA,
        ],
        compiler_params=pltpu.CompilerParams(needs_layout_passes=False),
    )(_sc_body)

    out = sc(sims_flat, inv_flat, patches_flat)
    return out.reshape(b, 8, d)[:, :k, :]


# speedup vs baseline: 1.1978x; 1.1978x over previous
import functools

import jax
import jax.numpy as jnp
from jax import lax
from jax.experimental import pallas as pl
from jax.experimental.pallas import tpu as pltpu
from jax.experimental.pallas import tpu_sc as plsc


def _tc_sims_kernel(cue_ref, patches_ref, sims_ref, inv_ref):
    cue = cue_ref[0]
    p = patches_ref[0]

    cue_n = cue / jnp.maximum(
        jnp.sqrt(jnp.sum(cue * cue, axis=1, keepdims=True)), 1e-12)
    inv = 1.0 / jnp.maximum(jnp.sqrt(jnp.sum(p * p, axis=1)), 1e-12)
    pn = p * inv[:, None]

    sims_ref[0] = jax.lax.dot_general(
        cue_n, pn, (((1,), (1,)), ((), ())),
        preferred_element_type=jnp.float32,
    )
    inv_ref[0, 0] = inv


def _dyn_gather(v, idx):
    return lax.gather(
        v, idx[:, None],
        dimension_numbers=lax.GatherDimensionNumbers(
            offset_dims=(), collapsed_slice_dims=(0,), start_index_map=(0,)),
        slice_sizes=(1,),
        mode=lax.GatherScatterMode.PROMISE_IN_BOUNDS)

def _sc_body(sims_hbm, inv_hbm, patches_hbm, out_hbm,
             sims_v, inv_v, idx_v, rows_v, acc_v, sem):
    nc = 2
    wid = lax.axis_index("s") * nc + lax.axis_index("c")

    def one_row(rep, carry):
        pair = wid + 32 * rep
        b = lax.div(pair, 5)
        k = lax.rem(pair, 5)
        r = b * 8 + k

        @pl.when(pair < 80)
        def _():
            pltpu.sync_copy(sims_hbm.at[r], sims_v)
            pltpu.sync_copy(inv_hbm.at[b], inv_v)

            lane = lax.iota(jnp.int32, 16)

            def chunk_step(j, c):
                cur_v, cur_i = c
                v = sims_v[pl.ds(j * 16, 16)]
                idx = lane + j * 16
                dv, di = plsc.sort_key_val(v, idx, descending=True)
                take = (dv > cur_v) | ((dv == cur_v) & (di < cur_i))
                nv, ni = plsc.sort_key_val(jnp.where(take, dv, cur_v),
                                           jnp.where(take, di, cur_i))
                return (nv, ni)

            cur_v, cur_i = lax.fori_loop(
                0, 256, chunk_step,
                (jnp.full((16,), -2.0, jnp.float32),
                 jnp.zeros((16,), jnp.int32)))

            cnt = jnp.zeros((16,), jnp.int32)
            for j in range(16):
                sel = jnp.full((16,), j, jnp.int32)
                vj = _dyn_gather(cur_v, sel)
                ij = _dyn_gather(cur_i, sel)
                beats = (vj > cur_v) | ((vj == cur_v) & (ij < cur_i))
                cnt = cnt + beats.astype(jnp.int32)
            _, top_i = plsc.sort_key_val(cnt, cur_i)
            inv9 = plsc.load_gather(inv_v, [top_i])
            wvec = inv9 * jnp.float32(1.0 / 9.0)

            idx_v[...] = top_i + b * 4096
            pltpu.async_copy(patches_hbm.at[idx_v], rows_v, sem).wait()

            def acc_chunk(c, carry):
                s = jnp.zeros((16,), jnp.float32)
                for j in range(9):
                    wj = _dyn_gather(wvec, jnp.full((16,), j, jnp.int32))
                    s = s + rows_v[j, pl.ds(c * 16, 16)] * wj
                acc_v[pl.ds(c * 16, 16)] = s
                return carry

            lax.fori_loop(0, 32, acc_chunk, 0)
            pltpu.sync_copy(acc_v, out_hbm.at[r])

        return carry

    lax.fori_loop(0, 3, one_row, 0)


def kernel(cue, patches):
    b, k, d = cue.shape
    n = patches.shape[1]
    cue_p = jnp.pad(cue, ((0, 0), (0, 8 - k), (0, 0)))

    sims, inv = pl.pallas_call(
        _tc_sims_kernel,
        grid=(b,),
        in_specs=[
            pl.BlockSpec((1, 8, d), lambda i: (i, 0, 0)),
            pl.BlockSpec((1, n, d), lambda i: (i, 0, 0)),
        ],
        out_specs=[
            pl.BlockSpec((1, 8, n), lambda i: (i, 0, 0)),
            pl.BlockSpec((1, 1, n), lambda i: (i, 0, 0)),
        ],
        out_shape=[
            jax.ShapeDtypeStruct((b, 8, n), jnp.float32),
            jax.ShapeDtypeStruct((b, 1, n), jnp.float32),
        ],
    )(cue_p, patches)

    sims_flat = sims.reshape(b * 8, n)
    inv_flat = inv.reshape(b, n)
    patches_flat = patches.reshape(b * n, d)

    mesh = plsc.VectorSubcoreMesh(core_axis_name="c", subcore_axis_name="s")
    sc = functools.partial(
        pl.kernel,
        mesh=mesh,
        out_type=jax.ShapeDtypeStruct((b * 8, d), jnp.float32),
        scratch_types=[
            pltpu.VMEM((n,), jnp.float32),
            pltpu.VMEM((n,), jnp.float32),
            pltpu.VMEM((16,), jnp.int32),
            pltpu.VMEM((16, d), jnp.float32),
            pltpu.VMEM((d,), jnp.float32),
            pltpu.SemaphoreType.DMA,
        ],
        compiler_params=pltpu.CompilerParams(needs_layout_passes=False),
    )(_sc_body)

    out = sc(sims_flat, inv_flat, patches_flat)
    return out.reshape(b, 8, d)[:, :k, :]


# speedup vs baseline: 1.2924x; 1.0790x over previous
import functools

import jax
import jax.numpy as jnp
from jax import lax
from jax.experimental import pallas as pl
from jax.experimental.pallas import tpu as pltpu
from jax.experimental.pallas import tpu_sc as plsc


def _tc_sims_kernel(cue_ref, patches_ref, sims_ref, inv_ref):
    cue = cue_ref[0]
    p = patches_ref[0]

    cue_n = cue / jnp.maximum(
        jnp.sqrt(jnp.sum(cue * cue, axis=1, keepdims=True)), 1e-12)

    sq = p * p
    s1 = sq[:, :256] + sq[:, 256:]
    s2 = s1[:, :128] + s1[:, 128:]
    ssum = jnp.sum(jnp.transpose(s2), axis=0, keepdims=True)
    inv_row = 1.0 / jnp.maximum(jnp.sqrt(ssum), 1e-12)
    inv_col = jnp.transpose(inv_row)
    pn = p * inv_col

    sims_ref[0] = jax.lax.dot_general(
        cue_n, pn, (((1,), (1,)), ((), ())),
        preferred_element_type=jnp.float32,
    )
    inv_ref[0] = inv_row


def _dyn_gather(v, idx):
    return lax.gather(
        v, idx[:, None],
        dimension_numbers=lax.GatherDimensionNumbers(
            offset_dims=(), collapsed_slice_dims=(0,), start_index_map=(0,)),
        slice_sizes=(1,),
        mode=lax.GatherScatterMode.PROMISE_IN_BOUNDS)

def _sc_body(sims_hbm, inv_hbm, patches_hbm, out_hbm,
             sims_v, inv_v, idx_v, rows_v, acc_v, sem):
    nc = 2
    wid = lax.axis_index("s") * nc + lax.axis_index("c")

    def one_row(rep, carry):
        pair = wid + 32 * rep
        b = lax.div(pair, 5)
        k = lax.rem(pair, 5)
        r = b * 8 + k

        @pl.when(pair < 80)
        def _():
            pltpu.sync_copy(sims_hbm.at[r], sims_v)
            pltpu.sync_copy(inv_hbm.at[b], inv_v)

            lane = lax.iota(jnp.int32, 16)

            def chunk_step(j, c):
                cur_v, cur_i = c
                v = sims_v[pl.ds(j * 16, 16)]
                idx = lane + j * 16
                dv, di = plsc.sort_key_val(v, idx, descending=True)
                take = (dv > cur_v) | ((dv == cur_v) & (di < cur_i))
                nv, ni = plsc.sort_key_val(jnp.where(take, dv, cur_v),
                                           jnp.where(take, di, cur_i))
                return (nv, ni)

            cur_v, cur_i = lax.fori_loop(
                0, 256, chunk_step,
                (jnp.full((16,), -2.0, jnp.float32),
                 jnp.zeros((16,), jnp.int32)))

            cnt = jnp.zeros((16,), jnp.int32)
            for j in range(16):
                sel = jnp.full((16,), j, jnp.int32)
                vj = _dyn_gather(cur_v, sel)
                ij = _dyn_gather(cur_i, sel)
                beats = (vj > cur_v) | ((vj == cur_v) & (ij < cur_i))
                cnt = cnt + beats.astype(jnp.int32)
            _, top_i = plsc.sort_key_val(cnt, cur_i)
            inv9 = plsc.load_gather(inv_v, [top_i])
            wvec = inv9 * jnp.float32(1.0 / 9.0)

            idx_v[...] = top_i + b * 4096
            pltpu.async_copy(patches_hbm.at[idx_v], rows_v, sem).wait()

            def acc_chunk(c, carry):
                s = jnp.zeros((16,), jnp.float32)
                for j in range(9):
                    wj = _dyn_gather(wvec, jnp.full((16,), j, jnp.int32))
                    s = s + rows_v[j, pl.ds(c * 16, 16)] * wj
                acc_v[pl.ds(c * 16, 16)] = s
                return carry

            lax.fori_loop(0, 32, acc_chunk, 0)
            pltpu.sync_copy(acc_v, out_hbm.at[r])

        return carry

    lax.fori_loop(0, 3, one_row, 0)


def kernel(cue, patches):
    b, k, d = cue.shape
    n = patches.shape[1]
    cue_p = jnp.pad(cue, ((0, 0), (0, 8 - k), (0, 0)))

    sims, inv = pl.pallas_call(
        _tc_sims_kernel,
        grid=(b,),
        in_specs=[
            pl.BlockSpec((1, 8, d), lambda i: (i, 0, 0)),
            pl.BlockSpec((1, n, d), lambda i: (i, 0, 0)),
        ],
        out_specs=[
            pl.BlockSpec((1, 8, n), lambda i: (i, 0, 0)),
            pl.BlockSpec((1, 1, n), lambda i: (i, 0, 0)),
        ],
        out_shape=[
            jax.ShapeDtypeStruct((b, 8, n), jnp.float32),
            jax.ShapeDtypeStruct((b, 1, n), jnp.float32),
        ],
    )(cue_p, patches)

    sims_flat = sims.reshape(b * 8, n)
    inv_flat = inv.reshape(b, n)
    patches_flat = patches.reshape(b * n, d)

    mesh = plsc.VectorSubcoreMesh(core_axis_name="c", subcore_axis_name="s")
    sc = functools.partial(
        pl.kernel,
        mesh=mesh,
        out_type=jax.ShapeDtypeStruct((b * 8, d), jnp.float32),
        scratch_types=[
            pltpu.VMEM((n,), jnp.float32),
            pltpu.VMEM((n,), jnp.float32),
            pltpu.VMEM((16,), jnp.int32),
            pltpu.VMEM((16, d), jnp.float32),
            pltpu.VMEM((d,), jnp.float32),
            pltpu.SemaphoreType.DMA,
        ],
        compiler_params=pltpu.CompilerParams(needs_layout_passes=False),
    )(_sc_body)

    out = sc(sims_flat, inv_flat, patches_flat)
    return out.reshape(b, 8, d)[:, :k, :]


# speedup vs baseline: 1.3147x; 1.0172x over previous
import functools

import jax
import jax.numpy as jnp
from jax import lax
from jax.experimental import pallas as pl
from jax.experimental.pallas import tpu as pltpu
from jax.experimental.pallas import tpu_sc as plsc


def _tc_sims_kernel(cue_ref, patches_ref, sims_ref, inv_ref):
    cue = cue_ref[0]
    p = patches_ref[0]

    cue_n = cue / jnp.maximum(
        jnp.sqrt(jnp.sum(cue * cue, axis=1, keepdims=True)), 1e-12)

    sq = p * p
    s1 = sq[:, :256] + sq[:, 256:]
    s2 = s1[:, :128] + s1[:, 128:]
    ssum = jnp.sum(jnp.transpose(s2), axis=0, keepdims=True)
    inv_row = 1.0 / jnp.maximum(jnp.sqrt(ssum), 1e-12)
    inv_col = jnp.transpose(inv_row)
    pn = p * inv_col

    sims_ref[0] = jax.lax.dot_general(
        cue_n, pn, (((1,), (1,)), ((), ())),
        preferred_element_type=jnp.float32,
    )
    inv_ref[0] = inv_row


def _dyn_gather(v, idx):
    return lax.gather(
        v, idx[:, None],
        dimension_numbers=lax.GatherDimensionNumbers(
            offset_dims=(), collapsed_slice_dims=(0,), start_index_map=(0,)),
        slice_sizes=(1,),
        mode=lax.GatherScatterMode.PROMISE_IN_BOUNDS)

def _sc_body(sims_hbm, inv_hbm, patches_hbm, out_hbm,
             sims_v, inv_v, idx_v, rows_v, acc_v, sem):
    nc = 2
    wid = lax.axis_index("s") * nc + lax.axis_index("c")

    def one_row(rep, carry):
        pair = wid + 32 * rep
        b = lax.div(pair, 5)
        k = lax.rem(pair, 5)
        r = b * 8 + k

        @pl.when(pair < 80)
        def _():
            pltpu.sync_copy(sims_hbm.at[r], sims_v)
            pltpu.sync_copy(inv_hbm.at[b], inv_v)

            lane = lax.iota(jnp.int32, 16)

            def chunk_step(j, c):
                v = sims_v[pl.ds(j * 16, 16)]
                i = lane + j * 16
                new = []
                for lvl in range(9):
                    lv, li = c[lvl]
                    take = v > lv
                    nv = jnp.where(take, v, lv)
                    ni = jnp.where(take, i, li)
                    v = jnp.where(take, lv, v)
                    i = jnp.where(take, li, i)
                    new.append((nv, ni))
                return tuple(new)

            ladder = lax.fori_loop(
                0, 256, chunk_step,
                tuple((jnp.full((16,), -2.0, jnp.float32),
                       jnp.full((16,), 0, jnp.int32)) for _ in range(9)))

            cur_v, cur_i = plsc.sort_key_val(ladder[0][0], ladder[0][1])
            for lvl in range(1, 9):
                dv, di = plsc.sort_key_val(ladder[lvl][0], ladder[lvl][1],
                                           descending=True)
                take = (dv > cur_v) | ((dv == cur_v) & (di < cur_i))
                cur_v, cur_i = plsc.sort_key_val(
                    jnp.where(take, dv, cur_v), jnp.where(take, di, cur_i))

            cnt = jnp.zeros((16,), jnp.int32)
            for j in range(16):
                sel = jnp.full((16,), j, jnp.int32)
                vj = _dyn_gather(cur_v, sel)
                ij = _dyn_gather(cur_i, sel)
                beats = (vj > cur_v) | ((vj == cur_v) & (ij < cur_i))
                cnt = cnt + beats.astype(jnp.int32)
            _, top_i = plsc.sort_key_val(cnt, cur_i)
            inv9 = plsc.load_gather(inv_v, [top_i])
            wvec = inv9 * jnp.float32(1.0 / 9.0)

            idx_v[...] = top_i + b * 4096
            pltpu.async_copy(patches_hbm.at[idx_v], rows_v, sem).wait()

            def acc_chunk(c, carry):
                s = jnp.zeros((16,), jnp.float32)
                for j in range(9):
                    wj = _dyn_gather(wvec, jnp.full((16,), j, jnp.int32))
                    s = s + rows_v[j, pl.ds(c * 16, 16)] * wj
                acc_v[pl.ds(c * 16, 16)] = s
                return carry

            lax.fori_loop(0, 32, acc_chunk, 0)
            pltpu.sync_copy(acc_v, out_hbm.at[r])

        return carry

    lax.fori_loop(0, 3, one_row, 0)


def kernel(cue, patches):
    b, k, d = cue.shape
    n = patches.shape[1]
    cue_p = jnp.pad(cue, ((0, 0), (0, 8 - k), (0, 0)))

    sims, inv = pl.pallas_call(
        _tc_sims_kernel,
        grid=(b,),
        in_specs=[
            pl.BlockSpec((1, 8, d), lambda i: (i, 0, 0)),
            pl.BlockSpec((1, n, d), lambda i: (i, 0, 0)),
        ],
        out_specs=[
            pl.BlockSpec((1, 8, n), lambda i: (i, 0, 0)),
            pl.BlockSpec((1, 1, n), lambda i: (i, 0, 0)),
        ],
        out_shape=[
            jax.ShapeDtypeStruct((b, 8, n), jnp.float32),
            jax.ShapeDtypeStruct((b, 1, n), jnp.float32),
        ],
    )(cue_p, patches)

    sims_flat = sims.reshape(b * 8, n)
    inv_flat = inv.reshape(b, n)
    patches_flat = patches.reshape(b * n, d)

    mesh = plsc.VectorSubcoreMesh(core_axis_name="c", subcore_axis_name="s")
    sc = functools.partial(
        pl.kernel,
        mesh=mesh,
        out_type=jax.ShapeDtypeStruct((b * 8, d), jnp.float32),
        scratch_types=[
            pltpu.VMEM((n,), jnp.float32),
            pltpu.VMEM((n,), jnp.float32),
            pltpu.VMEM((16,), jnp.int32),
            pltpu.VMEM((16, d), jnp.float32),
            pltpu.VMEM((d,), jnp.float32),
            pltpu.SemaphoreType.DMA,
        ],
        compiler_params=pltpu.CompilerParams(needs_layout_passes=False),
    )(_sc_body)

    out = sc(sims_flat, inv_flat, patches_flat)
    return out.reshape(b, 8, d)[:, :k, :]
